# baseline (speedup 1.0000x reference)
import functools

import jax
import jax.numpy as jnp
from jax import lax
from jax.experimental import pallas as pl
from jax.experimental.pallas import tpu as pltpu
from jax.experimental.pallas import tpu_sc as plsc

DEPTH = 12
F = 128
N = 10000
E = 320000

NC = 2
NS = 16
NW = NC * NS
EPW = E // NW
B = 80
K = EPW // B
NPAD = 10240
NPT = NPAD // NS
FH = F // 2
EPT = E // NS
B2 = 32
NH = 5
K2 = EPT // (NH * B2)
NPB = 5120
NPB2 = 5248
NPT2 = NPB2 // NS
NZ = N + 8

BE = 2000
GE = E // BE


def _sc_mesh():
    return plsc.VectorSubcoreMesh(
        core_axis_name="c", subcore_axis_name="s",
        num_cores=NC, num_subcores=NS)


def _sc_gather_sum(x3, x4, idx_s3, idx_d3):
    @functools.partial(
        pl.kernel,
        out_type=jax.ShapeDtypeStruct((E, F), jnp.float32),
        mesh=_sc_mesh(),
        scratch_types=[
            pltpu.VMEM((K, B), jnp.int32),
            pltpu.VMEM((K, B), jnp.int32),
            [pltpu.VMEM((B, F), jnp.float32)] * 2,
            [pltpu.VMEM((B, F), jnp.float32)] * 2,
            [pltpu.VMEM((B, F), jnp.float32)] * 2,
            [pltpu.SemaphoreType.DMA] * 2,
            [pltpu.SemaphoreType.DMA] * 2,
        ],
    )
    def k(x3_hbm, x4_hbm, idxs_hbm, idxd_hbm, s_hbm, idxs_v, idxd_v,
          b3, b4, bo, sg, sw):
        wid = lax.axis_index("s") * NC + lax.axis_index("c")
        base = wid * EPW
        pltpu.sync_copy(idxs_hbm.at[wid], idxs_v)
        pltpu.sync_copy(idxd_hbm.at[wid], idxd_v)

        def gather(j, t):
            pltpu.async_copy(x3_hbm.at[idxs_v.at[j]], b3[t], sg[t])
            pltpu.async_copy(x4_hbm.at[idxd_v.at[j]], b4[t], sg[t])

        def process(j, t):
            pltpu.make_async_copy(x3_hbm.at[idxs_v.at[0]], b3[t], sg[t]).wait()
            pltpu.make_async_copy(x4_hbm.at[idxd_v.at[0]], b4[t], sg[t]).wait()

            @pl.when(j >= 2)
            def _():
                pltpu.make_async_copy(bo[t], s_hbm.at[pl.ds(0, B)],
                                      sw[t]).wait()

            def row(i, _):
                for l in range(F // 16):
                    sl = pl.ds(l * 16, 16)
                    bo[t][i, sl] = b3[t][i, sl] + b4[t][i, sl]
                return 0

            lax.fori_loop(0, B, row, 0)
            pltpu.async_copy(bo[t], s_hbm.at[pl.ds(base + j * B, B)], sw[t])

            @pl.when(j + 2 < K)
            def _():
                gather(j + 2, t)

        gather(0, 0)
        gather(1, 1)

        def pair(p, _):
            process(2 * p, 0)
            process(2 * p + 1, 1)
            return 0

        lax.fori_loop(0, K // 2, pair, 0)
        process(jnp.int32(K - 1), 0)
        pltpu.make_async_copy(bo[0], s_hbm.at[pl.ds(0, B)], sw[0]).wait()
        pltpu.make_async_copy(bo[1], s_hbm.at[pl.ds(0, B)], sw[1]).wait()

    return k(x3, x4, idx_s3, idx_d3)


def _sc_gate_scatter(x2z, gate, idx_s2, idx_d2, zeros_n, order_dep):
    @functools.partial(
        pl.kernel,
        out_type=jax.ShapeDtypeStruct((NC, NPB2, F), jnp.float32),
        mesh=_sc_mesh(),
        scratch_types=[
            pltpu.VMEM((K2, B2), jnp.int32),
            pltpu.VMEM((K2, B2), jnp.int32),
            [pltpu.VMEM((B2, F), jnp.float32)] * 2,
            [pltpu.VMEM((B2, F), jnp.float32)] * 2,
            [pltpu.VMEM((B2, F), jnp.float32)] * 2,
            pltpu.VMEM_SHARED((NPB2, F), jnp.float32),
            [pltpu.SemaphoreType.DMA] * 2,
            [pltpu.SemaphoreType.DMA] * 2,
        ],
    )
    def k(x2_hbm, g_hbm, idxs_hbm, idxd_hbm, z_hbm, dep_hbm, acc_hbm,
          idxs_v, idxd_v, bx, bw, bs, acc_sh, sg, ss):
        del dep_hbm
        cid = lax.axis_index("c")
        sid = lax.axis_index("s")
        coff = cid * NPB
        rows = pl.ds(sid * NPT2, NPT2)
        pltpu.sync_copy(z_hbm.at[rows], acc_sh.at[rows])
        plsc.subcore_barrier()

        def phase(h):
            base = sid * EPT + h * (K2 * B2)
            pltpu.sync_copy(idxs_hbm.at[sid, h], idxs_v)
            pltpu.sync_copy(idxd_hbm.at[sid, h], idxd_v)

            def adj(r, _):
                for l in range(B2 // 16):
                    sl = pl.ds(l * 16, 16)
                    sv = idxs_v[r, sl]
                    r0 = sv - coff
                    valid = (r0 >= 0) & (r0 < NPB)
                    idxs_v[r, sl] = jnp.where(valid, r0,
                                              NPB + (sv & (NPB2 - NPB - 1)))
                return 0

            lax.fori_loop(0, K2, adj, 0)

            def gather(j, t):
                pltpu.async_copy(x2_hbm.at[idxd_v.at[j]], bx[t], sg[t])
                pltpu.async_copy(g_hbm.at[pl.ds(base + j * B2, B2)],
                                 bw[t], sg[t])

            def process(j, t):
                pltpu.make_async_copy(x2_hbm.at[idxd_v.at[0]],
                                      bx[t], sg[t]).wait()
                pltpu.make_async_copy(g_hbm.at[pl.ds(0, B2)],
                                      bw[t], sg[t]).wait()

                @pl.when(j >= 2)
                def _():
                    pltpu.make_async_copy(bs[t], acc_sh.at[idxs_v.at[0]],
                                          ss[t]).wait()

                def row(i, _):
                    for l in range(F // 16):
                        sl = pl.ds(l * 16, 16)
                        bs[t][i, sl] = bx[t][i, sl] * bw[t][i, sl]
                    return 0

                lax.fori_loop(0, B2, row, 0)

                @pl.when(j + 2 < K2)
                def _():
                    gather(j + 2, t)

                pltpu.async_copy(bs[t], acc_sh.at[idxs_v.at[j]],
                                 ss[t], add=True)

            gather(0, 0)
            gather(1, 1)

            def pair(p, _):
                process(2 * p, 0)
                process(2 * p + 1, 1)
                return 0

            lax.fori_loop(0, K2 // 2, pair, 0)
            process(jnp.int32(K2 - 1), 0)
            pltpu.make_async_copy(bs[0], acc_sh.at[idxs_v.at[0]],
                                  ss[0]).wait()
            pltpu.make_async_copy(bs[1], acc_sh.at[idxs_v.at[0]],
                                  ss[1]).wait()

        for h in range(NH):
            phase(h)
        plsc.subcore_barrier()
        pltpu.sync_copy(acc_sh.at[rows], acc_hbm.at[cid, rows])

    return k(x2z, gate, idx_s2, idx_d2, zeros_n, order_dep)


def _tc_init_node(x, W, b):
    def body(x_ref, w_ref, b_ref, o_ref):
        h = jnp.dot(x_ref[...], w_ref[...],
                    preferred_element_type=jnp.float32) + b_ref[...]
        o_ref[...] = h * jax.nn.sigmoid(h)

    return pl.pallas_call(
        body,
        out_shape=jax.ShapeDtypeStruct((N, F), jnp.float32),
    )(x, W, b.reshape(1, F))


def _tc_init_edge(ea, W, b):
    def body(ea_ref, w_ref, b_ref, o_ref, g_ref):
        h = jnp.dot(ea_ref[...], w_ref[...],
                    preferred_element_type=jnp.float32) + b_ref[...]
        o = h * jax.nn.sigmoid(h)
        o_ref[...] = o
        g_ref[...] = jax.nn.sigmoid(o)

    ef = ea.shape[1]
    return pl.pallas_call(
        body,
        grid=(GE,),
        in_specs=[
            pl.BlockSpec((BE, ef), lambda i: (i, 0)),
            pl.BlockSpec((ef, F), lambda i: (0, 0)),
            pl.BlockSpec((1, F), lambda i: (0, 0)),
        ],
        out_specs=(
            pl.BlockSpec((BE, F), lambda i: (i, 0)),
            pl.BlockSpec((BE, F), lambda i: (i, 0)),
        ),
        out_shape=(
            jax.ShapeDtypeStruct((E, F), jnp.float32),
            jax.ShapeDtypeStruct((E, F), jnp.float32),
        ),
    )(ea, W, b.reshape(1, F))


def _tc_node_mm(x, Wcat, bcat):
    def body(x_ref, w_ref, b_ref, o1, o2, o4, o3):
        h = jnp.dot(x_ref[...], w_ref[...],
                    preferred_element_type=jnp.float32) + b_ref[...]
        o1[...] = h[:, 0 * F:1 * F]
        o2[...] = h[:, 1 * F:2 * F]
        o4[...] = h[:, 2 * F:3 * F]
        o3[...] = h[:, 3 * F:4 * F]

    sh = jax.ShapeDtypeStruct((N, F), jnp.float32)
    return pl.pallas_call(
        body,
        out_shape=(sh, sh, sh, sh),
    )(x, Wcat, bcat.reshape(1, 4 * F))


def _tc_edge_pass1(w, s, E0Wi, E0bi):
    def body(w_ref, s_ref, ew_ref, eb_ref, st_ref):
        h = jnp.dot(w_ref[...], ew_ref[...],
                    preferred_element_type=jnp.float32)
        h = h + eb_ref[...] + s_ref[...]

        @pl.when(pl.program_id(0) == 0)
        def _():
            st_ref[...] = jnp.zeros_like(st_ref)

        st_ref[0:1, :] += jnp.sum(h, axis=0, keepdims=True)
        st_ref[1:2, :] += jnp.sum(h * h, axis=0, keepdims=True)

    return pl.pallas_call(
        body,
        grid=(GE,),
        in_specs=[
            pl.BlockSpec((BE, F), lambda i: (i, 0)),
            pl.BlockSpec((BE, F), lambda i: (i, 0)),
            pl.BlockSpec((F, F), lambda i: (0, 0)),
            pl.BlockSpec((1, F), lambda i: (0, 0)),
        ],
        out_specs=pl.BlockSpec((8, F), lambda i: (0, 0)),
        out_shape=jax.ShapeDtypeStruct((8, F), jnp.float32),
    )(w, s, E0Wi, E0bi.reshape(1, F))


def _tc_inv_deg(cnt):
    def body(c_ref, o_ref):
        c = jnp.concatenate([c_ref[0, :NPB], c_ref[1, :N - NPB]], axis=0)
        o_ref[...] = 1.0 / jnp.maximum(c, 1.0)

    return pl.pallas_call(
        body,
        out_shape=jax.ShapeDtypeStruct((N, F), jnp.float32),
    )(cnt)


def _tc_node_update(x0, x1, acc, inv, g, b):
    def body(x0_ref, x1_ref, a_ref, i_ref, g_ref, b_ref, o_ref):
        asum = jnp.concatenate(
            [a_ref[0, :NPB], a_ref[1, :N - NPB]], axis=0)
        t = x1_ref[...] + asum * i_ref[...]
        m = jnp.mean(t, axis=0, keepdims=True)
        v = jnp.mean(t * t, axis=0, keepdims=True) - m * m
        y = (t - m) * lax.rsqrt(v + 1e-5) * g_ref[...] + b_ref[...]
        o_ref[...] = x0_ref[...] + y * jax.nn.sigmoid(y)

    return pl.pallas_call(
        body,
        out_shape=jax.ShapeDtypeStruct((N, F), jnp.float32),
    )(x0, x1, acc, inv, g.reshape(1, F), b.reshape(1, F))


def _tc_edge_update(w0, s, E0Wi, E0bi, st, g, b, want_gate):
    def body(w_ref, s_ref, ew_ref, eb_ref, st_ref, g_ref, b_ref,
             o_ref, *rest):
        h = jnp.dot(w_ref[...], ew_ref[...],
                    preferred_element_type=jnp.float32)
        h = h + eb_ref[...] + s_ref[...]
        m = st_ref[0:1, :] * (1.0 / E)
        v = st_ref[1:2, :] * (1.0 / E) - m * m
        y = (h - m) * lax.rsqrt(v + 1e-5) * g_ref[...] + b_ref[...]
        o = w_ref[...] + y * jax.nn.sigmoid(y)
        o_ref[...] = o
        if rest:
            rest[0][...] = jax.nn.sigmoid(o)

    out_specs = [pl.BlockSpec((BE, F), lambda i: (i, 0))]
    out_shape = [jax.ShapeDtypeStruct((E, F), jnp.float32)]
    if want_gate:
        out_specs.append(pl.BlockSpec((BE, F), lambda i: (i, 0)))
        out_shape.append(jax.ShapeDtypeStruct((E, F), jnp.float32))
    res = pl.pallas_call(
        body,
        grid=(GE,),
        in_specs=[
            pl.BlockSpec((BE, F), lambda i: (i, 0)),
            pl.BlockSpec((BE, F), lambda i: (i, 0)),
            pl.BlockSpec((F, F), lambda i: (0, 0)),
            pl.BlockSpec((1, F), lambda i: (0, 0)),
            pl.BlockSpec((8, F), lambda i: (0, 0)),
            pl.BlockSpec((1, F), lambda i: (0, 0)),
            pl.BlockSpec((1, F), lambda i: (0, 0)),
        ],
        out_specs=tuple(out_specs),
        out_shape=tuple(out_shape),
    )(w0, s, E0Wi, E0bi.reshape(1, F), st, g.reshape(1, F),
      b.reshape(1, F))
    return res if want_gate else (res[0], None)


def kernel(x, edge_index, edge_attr, v_lin0_W, v_lin0_b, V1_W, V1_b, V2_W,
           V2_b, V3_W, V3_b, V4_W, V4_b, vbn_g, vbn_b, e_lin0_W, e_lin0_b,
           E0_W, E0_b, ebn_g, ebn_b):
    src = edge_index[0]
    dst = edge_index[1]
    idx_s3 = src.reshape(NW, K, B)
    idx_d3 = dst.reshape(NW, K, B)
    idx_s2 = src.reshape(NS, NH, K2, B2)
    idx_d2 = dst.reshape(NS, NH, K2, B2)
    zeros_b2 = jnp.zeros((NPB2, F), jnp.float32)
    ones_z = jnp.ones((N, F), jnp.float32)
    ones_e = jnp.ones((E, F), jnp.float32)

    Wcat = jnp.concatenate([V1_W, V2_W, V4_W, V3_W], axis=2)
    bcat = jnp.concatenate([V1_b, V2_b, V4_b, V3_b], axis=1)

    dep0 = jnp.zeros((8, F), jnp.float32)
    cnt = _sc_gate_scatter(ones_z, ones_e, idx_s2, idx_d2, zeros_b2, dep0)
    inv = _tc_inv_deg(cnt)

    xc = _tc_init_node(x, v_lin0_W, v_lin0_b)
    wc, gc = _tc_init_edge(edge_attr, e_lin0_W, e_lin0_b)

    x1, x2z, x4, x3 = _tc_node_mm(xc, Wcat[0], bcat[0])
    s = _sc_gather_sum(x3, x4, idx_s3, idx_d3)
    acc = _sc_gate_scatter(x2z, gc, idx_s2, idx_d2, zeros_b2, s[:8])
    for i in range(DEPTH):
        st = _tc_edge_pass1(wc, s, E0_W[i], E0_b[i])
        xc = _tc_node_update(xc, x1, acc, inv, vbn_g[i], vbn_b[i])
        if i + 1 < DEPTH:
            x1, x2z, x4, x3 = _tc_node_mm(xc, Wcat[i + 1], bcat[i + 1])
            sn = _sc_gather_sum(x3, x4, idx_s3, idx_d3)
            wc, gc = _tc_edge_update(wc, s, E0_W[i], E0_b[i], st,
                                     ebn_g[i], ebn_b[i], want_gate=True)
            acc = _sc_gate_scatter(x2z, gc, idx_s2, idx_d2, zeros_b2, sn[:8])
            s = sn
        else:
            wc, _ = _tc_edge_update(wc, s, E0_W[i], E0_b[i], st,
                                    ebn_g[i], ebn_b[i], want_gate=False)
    return (xc, wc)


# speedup vs baseline: 1.0576x; 1.0576x over previous
import functools

import jax
import jax.numpy as jnp
from jax import lax
from jax.experimental import pallas as pl
from jax.experimental.pallas import tpu as pltpu
from jax.experimental.pallas import tpu_sc as plsc

DEPTH = 12
F = 128
N = 10000
E = 320000

NC = 2
NS = 16
NW = NC * NS
EPW = E // NW
B = 80
K = EPW // B
NPAD = 10240
NPT = NPAD // NS
FH = F // 2
EPT = E // NS
B2 = 32
NH = 5
K2 = EPT // (NH * B2)
NPB = 5120
NPB2 = 5248
NPT2 = NPB2 // NS
NZ = N + 8

BE = 4000
GE = E // BE


def _sc_mesh():
    return plsc.VectorSubcoreMesh(
        core_axis_name="c", subcore_axis_name="s",
        num_cores=NC, num_subcores=NS)


def _sc_gather_sum(x3, x4, idx_s3, idx_d3):
    @functools.partial(
        pl.kernel,
        out_type=jax.ShapeDtypeStruct((E, F), jnp.float32),
        mesh=_sc_mesh(),
        scratch_types=[
            pltpu.VMEM((K, B), jnp.int32),
            pltpu.VMEM((K, B), jnp.int32),
            [pltpu.VMEM((B, F), jnp.float32)] * 2,
            [pltpu.VMEM((B, F), jnp.float32)] * 2,
            [pltpu.VMEM((B, F), jnp.float32)] * 2,
            [pltpu.SemaphoreType.DMA] * 2,
            [pltpu.SemaphoreType.DMA] * 2,
        ],
    )
    def k(x3_hbm, x4_hbm, idxs_hbm, idxd_hbm, s_hbm, idxs_v, idxd_v,
          b3, b4, bo, sg, sw):
        wid = lax.axis_index("s") * NC + lax.axis_index("c")
        base = wid * EPW
        pltpu.sync_copy(idxs_hbm.at[wid], idxs_v)
        pltpu.sync_copy(idxd_hbm.at[wid], idxd_v)

        def gather(j, t):
            pltpu.async_copy(x3_hbm.at[idxs_v.at[j]], b3[t], sg[t])
            pltpu.async_copy(x4_hbm.at[idxd_v.at[j]], b4[t], sg[t])

        def process(j, t):
            pltpu.make_async_copy(x3_hbm.at[idxs_v.at[0]], b3[t], sg[t]).wait()
            pltpu.make_async_copy(x4_hbm.at[idxd_v.at[0]], b4[t], sg[t]).wait()

            @pl.when(j >= 2)
            def _():
                pltpu.make_async_copy(bo[t], s_hbm.at[pl.ds(0, B)],
                                      sw[t]).wait()

            def row(i, _):
                for l in range(F // 16):
                    sl = pl.ds(l * 16, 16)
                    bo[t][i, sl] = b3[t][i, sl] + b4[t][i, sl]
                return 0

            lax.fori_loop(0, B, row, 0)
            pltpu.async_copy(bo[t], s_hbm.at[pl.ds(base + j * B, B)], sw[t])

            @pl.when(j + 2 < K)
            def _():
                gather(j + 2, t)

        gather(0, 0)
        gather(1, 1)

        def pair(p, _):
            process(2 * p, 0)
            process(2 * p + 1, 1)
            return 0

        lax.fori_loop(0, K // 2, pair, 0)
        process(jnp.int32(K - 1), 0)
        pltpu.make_async_copy(bo[0], s_hbm.at[pl.ds(0, B)], sw[0]).wait()
        pltpu.make_async_copy(bo[1], s_hbm.at[pl.ds(0, B)], sw[1]).wait()

    return k(x3, x4, idx_s3, idx_d3)


def _sc_gate_scatter(x2z, gate, idx_s2, idx_d2, zeros_n, order_dep):
    @functools.partial(
        pl.kernel,
        out_type=jax.ShapeDtypeStruct((NC, NPB2, F), jnp.float32),
        mesh=_sc_mesh(),
        scratch_types=[
            pltpu.VMEM((K2, B2), jnp.int32),
            pltpu.VMEM((K2, B2), jnp.int32),
            [pltpu.VMEM((B2, F), jnp.float32)] * 2,
            [pltpu.VMEM((B2, F), jnp.float32)] * 2,
            [pltpu.VMEM((B2, F), jnp.float32)] * 2,
            pltpu.VMEM_SHARED((NPB2, F), jnp.float32),
            [pltpu.SemaphoreType.DMA] * 2,
            [pltpu.SemaphoreType.DMA] * 2,
        ],
    )
    def k(x2_hbm, g_hbm, idxs_hbm, idxd_hbm, z_hbm, dep_hbm, acc_hbm,
          idxs_v, idxd_v, bx, bw, bs, acc_sh, sg, ss):
        del dep_hbm
        cid = lax.axis_index("c")
        sid = lax.axis_index("s")
        coff = cid * NPB
        rows = pl.ds(sid * NPT2, NPT2)
        pltpu.sync_copy(z_hbm.at[rows], acc_sh.at[rows])
        plsc.subcore_barrier()

        def phase(h):
            base = sid * EPT + h * (K2 * B2)
            pltpu.sync_copy(idxs_hbm.at[sid, h], idxs_v)
            pltpu.sync_copy(idxd_hbm.at[sid, h], idxd_v)

            def adj(r, _):
                for l in range(B2 // 16):
                    sl = pl.ds(l * 16, 16)
                    sv = idxs_v[r, sl]
                    r0 = sv - coff
                    valid = (r0 >= 0) & (r0 < NPB)
                    idxs_v[r, sl] = jnp.where(valid, r0,
                                              NPB + (sv & (NPB2 - NPB - 1)))
                return 0

            lax.fori_loop(0, K2, adj, 0)

            def gather(j, t):
                pltpu.async_copy(x2_hbm.at[idxd_v.at[j]], bx[t], sg[t])
                pltpu.async_copy(g_hbm.at[pl.ds(base + j * B2, B2)],
                                 bw[t], sg[t])

            def process(j, t):
                pltpu.make_async_copy(x2_hbm.at[idxd_v.at[0]],
                                      bx[t], sg[t]).wait()
                pltpu.make_async_copy(g_hbm.at[pl.ds(0, B2)],
                                      bw[t], sg[t]).wait()

                @pl.when(j >= 2)
                def _():
                    pltpu.make_async_copy(bs[t], acc_sh.at[idxs_v.at[0]],
                                          ss[t]).wait()

                def row(i, _):
                    for l in range(F // 16):
                        sl = pl.ds(l * 16, 16)
                        bs[t][i, sl] = bx[t][i, sl] * bw[t][i, sl]
                    return 0

                lax.fori_loop(0, B2, row, 0)

                @pl.when(j + 2 < K2)
                def _():
                    gather(j + 2, t)

                pltpu.async_copy(bs[t], acc_sh.at[idxs_v.at[j]],
                                 ss[t], add=True)

            gather(0, 0)
            gather(1, 1)

            def pair(p, _):
                process(2 * p, 0)
                process(2 * p + 1, 1)
                return 0

            lax.fori_loop(0, K2 // 2, pair, 0)
            process(jnp.int32(K2 - 1), 0)
            pltpu.make_async_copy(bs[0], acc_sh.at[idxs_v.at[0]],
                                  ss[0]).wait()
            pltpu.make_async_copy(bs[1], acc_sh.at[idxs_v.at[0]],
                                  ss[1]).wait()

        for h in range(NH):
            phase(h)
        plsc.subcore_barrier()
        pltpu.sync_copy(acc_sh.at[rows], acc_hbm.at[cid, rows])

    return k(x2z, gate, idx_s2, idx_d2, zeros_n, order_dep)


def _tc_init_node(x, W, b):
    def body(x_ref, w_ref, b_ref, o_ref):
        h = jnp.dot(x_ref[...], w_ref[...],
                    preferred_element_type=jnp.float32) + b_ref[...]
        o_ref[...] = h * jax.nn.sigmoid(h)

    return pl.pallas_call(
        body,
        out_shape=jax.ShapeDtypeStruct((N, F), jnp.float32),
    )(x, W, b.reshape(1, F))


def _tc_init_edge(ea, W, b):
    def body(ea_ref, w_ref, b_ref, o_ref, g_ref):
        h = jnp.dot(ea_ref[...], w_ref[...],
                    preferred_element_type=jnp.float32) + b_ref[...]
        o = h * jax.nn.sigmoid(h)
        o_ref[...] = o
        g_ref[...] = jax.nn.sigmoid(o)

    ef = ea.shape[1]
    return pl.pallas_call(
        body,
        grid=(GE,),
        in_specs=[
            pl.BlockSpec((BE, ef), lambda i: (i, 0)),
            pl.BlockSpec((ef, F), lambda i: (0, 0)),
            pl.BlockSpec((1, F), lambda i: (0, 0)),
        ],
        out_specs=(
            pl.BlockSpec((BE, F), lambda i: (i, 0)),
            pl.BlockSpec((BE, F), lambda i: (i, 0)),
        ),
        out_shape=(
            jax.ShapeDtypeStruct((E, F), jnp.float32),
            jax.ShapeDtypeStruct((E, F), jnp.float32),
        ),
    )(ea, W, b.reshape(1, F))


def _tc_node_mm(x, Wcat, bcat):
    def body(x_ref, w_ref, b_ref, o1, o2, o4, o3):
        h = jnp.dot(x_ref[...], w_ref[...],
                    preferred_element_type=jnp.float32) + b_ref[...]
        o1[...] = h[:, 0 * F:1 * F]
        o2[...] = h[:, 1 * F:2 * F]
        o4[...] = h[:, 2 * F:3 * F]
        o3[...] = h[:, 3 * F:4 * F]

    sh = jax.ShapeDtypeStruct((N, F), jnp.float32)
    return pl.pallas_call(
        body,
        out_shape=(sh, sh, sh, sh),
    )(x, Wcat, bcat.reshape(1, 4 * F))


def _tc_edge_pass1(w, s, E0Wi, E0bi):
    def body(w_ref, s_ref, ew_ref, eb_ref, st_ref):
        h = jnp.dot(w_ref[...], ew_ref[...],
                    preferred_element_type=jnp.float32)
        h = h + eb_ref[...] + s_ref[...]

        @pl.when(pl.program_id(0) == 0)
        def _():
            st_ref[...] = jnp.zeros_like(st_ref)

        st_ref[0:1, :] += jnp.sum(h, axis=0, keepdims=True)
        st_ref[1:2, :] += jnp.sum(h * h, axis=0, keepdims=True)

    return pl.pallas_call(
        body,
        grid=(GE,),
        in_specs=[
            pl.BlockSpec((BE, F), lambda i: (i, 0)),
            pl.BlockSpec((BE, F), lambda i: (i, 0)),
            pl.BlockSpec((F, F), lambda i: (0, 0)),
            pl.BlockSpec((1, F), lambda i: (0, 0)),
        ],
        out_specs=pl.BlockSpec((8, F), lambda i: (0, 0)),
        out_shape=jax.ShapeDtypeStruct((8, F), jnp.float32),
    )(w, s, E0Wi, E0bi.reshape(1, F))


def _tc_inv_deg(cnt):
    def body(c_ref, o_ref):
        c = jnp.concatenate([c_ref[0, :NPB], c_ref[1, :N - NPB]], axis=0)
        o_ref[...] = 1.0 / jnp.maximum(c, 1.0)

    return pl.pallas_call(
        body,
        out_shape=jax.ShapeDtypeStruct((N, F), jnp.float32),
    )(cnt)


def _tc_node_update(x0, x1, acc, inv, g, b):
    def body(x0_ref, x1_ref, a_ref, i_ref, g_ref, b_ref, o_ref):
        asum = jnp.concatenate(
            [a_ref[0, :NPB], a_ref[1, :N - NPB]], axis=0)
        t = x1_ref[...] + asum * i_ref[...]
        m = jnp.mean(t, axis=0, keepdims=True)
        v = jnp.mean(t * t, axis=0, keepdims=True) - m * m
        y = (t - m) * lax.rsqrt(v + 1e-5) * g_ref[...] + b_ref[...]
        o_ref[...] = x0_ref[...] + y * jax.nn.sigmoid(y)

    return pl.pallas_call(
        body,
        out_shape=jax.ShapeDtypeStruct((N, F), jnp.float32),
    )(x0, x1, acc, inv, g.reshape(1, F), b.reshape(1, F))


def _tc_edge_update(w0, s, E0Wi, E0bi, st, g, b, want_gate):
    def body(w_ref, s_ref, ew_ref, eb_ref, st_ref, g_ref, b_ref,
             o_ref, *rest):
        h = jnp.dot(w_ref[...], ew_ref[...],
                    preferred_element_type=jnp.float32)
        h = h + eb_ref[...] + s_ref[...]
        m = st_ref[0:1, :] * (1.0 / E)
        v = st_ref[1:2, :] * (1.0 / E) - m * m
        y = (h - m) * lax.rsqrt(v + 1e-5) * g_ref[...] + b_ref[...]
        o = w_ref[...] + y * jax.nn.sigmoid(y)
        o_ref[...] = o
        if rest:
            rest[0][...] = jax.nn.sigmoid(o)

    out_specs = [pl.BlockSpec((BE, F), lambda i: (i, 0))]
    out_shape = [jax.ShapeDtypeStruct((E, F), jnp.float32)]
    if want_gate:
        out_specs.append(pl.BlockSpec((BE, F), lambda i: (i, 0)))
        out_shape.append(jax.ShapeDtypeStruct((E, F), jnp.float32))
    res = pl.pallas_call(
        body,
        grid=(GE,),
        in_specs=[
            pl.BlockSpec((BE, F), lambda i: (i, 0)),
            pl.BlockSpec((BE, F), lambda i: (i, 0)),
            pl.BlockSpec((F, F), lambda i: (0, 0)),
            pl.BlockSpec((1, F), lambda i: (0, 0)),
            pl.BlockSpec((8, F), lambda i: (0, 0)),
            pl.BlockSpec((1, F), lambda i: (0, 0)),
            pl.BlockSpec((1, F), lambda i: (0, 0)),
        ],
        out_specs=tuple(out_specs),
        out_shape=tuple(out_shape),
    )(w0, s, E0Wi, E0bi.reshape(1, F), st, g.reshape(1, F),
      b.reshape(1, F))
    return res if want_gate else (res[0], None)


def kernel(x, edge_index, edge_attr, v_lin0_W, v_lin0_b, V1_W, V1_b, V2_W,
           V2_b, V3_W, V3_b, V4_W, V4_b, vbn_g, vbn_b, e_lin0_W, e_lin0_b,
           E0_W, E0_b, ebn_g, ebn_b):
    src = edge_index[0]
    dst = edge_index[1]
    idx_s3 = src.reshape(NW, K, B)
    idx_d3 = dst.reshape(NW, K, B)
    idx_s2 = src.reshape(NS, NH, K2, B2)
    idx_d2 = dst.reshape(NS, NH, K2, B2)
    zeros_b2 = jnp.zeros((NPB2, F), jnp.float32)
    ones_z = jnp.ones((N, F), jnp.float32)
    ones_e = jnp.ones((E, F), jnp.float32)

    Wcat = jnp.concatenate([V1_W, V2_W, V4_W, V3_W], axis=2)
    bcat = jnp.concatenate([V1_b, V2_b, V4_b, V3_b], axis=1)

    dep0 = jnp.zeros((8, F), jnp.float32)
    cnt = _sc_gate_scatter(ones_z, ones_e, idx_s2, idx_d2, zeros_b2, dep0)
    inv = _tc_inv_deg(cnt)

    xc = _tc_init_node(x, v_lin0_W, v_lin0_b)
    wc, gc = _tc_init_edge(edge_attr, e_lin0_W, e_lin0_b)

    x1, x2z, x4, x3 = _tc_node_mm(xc, Wcat[0], bcat[0])
    s = _sc_gather_sum(x3, x4, idx_s3, idx_d3)
    acc = _sc_gate_scatter(x2z, gc, idx_s2, idx_d2, zeros_b2, s[:8])
    for i in range(DEPTH):
        st = _tc_edge_pass1(wc, s, E0_W[i], E0_b[i])
        xc = _tc_node_update(xc, x1, acc, inv, vbn_g[i], vbn_b[i])
        if i + 1 < DEPTH:
            x1, x2z, x4, x3 = _tc_node_mm(xc, Wcat[i + 1], bcat[i + 1])
            sn = _sc_gather_sum(x3, x4, idx_s3, idx_d3)
            wc, gc = _tc_edge_update(wc, s, E0_W[i], E0_b[i], st,
                                     ebn_g[i], ebn_b[i], want_gate=True)
            acc = _sc_gate_scatter(x2z, gc, idx_s2, idx_d2, zeros_b2, sn[:8])
            s = sn
        else:
            wc, _ = _tc_edge_update(wc, s, E0_W[i], E0_b[i], st,
                                    ebn_g[i], ebn_b[i], want_gate=False)
    return (xc, wc)


# speedup vs baseline: 1.0657x; 1.0077x over previous
import functools

import jax
import jax.numpy as jnp
from jax import lax
from jax.experimental import pallas as pl
from jax.experimental.pallas import tpu as pltpu
from jax.experimental.pallas import tpu_sc as plsc

DEPTH = 12
F = 128
N = 10000
E = 320000

NC = 2
NS = 16
NW = NC * NS
EPW = E // NW
B = 80
K = EPW // B
NPAD = 10240
NPT = NPAD // NS
FH = F // 2
EPT = E // NS
B2 = 32
NH = 5
K2 = EPT // (NH * B2)
NPB = 5120
NPB2 = 5248
NPT2 = NPB2 // NS
NZ = N + 8

BE = 8000
GE = E // BE


def _sc_mesh():
    return plsc.VectorSubcoreMesh(
        core_axis_name="c", subcore_axis_name="s",
        num_cores=NC, num_subcores=NS)


def _sc_gather_sum(x3, x4, idx_s3, idx_d3):
    @functools.partial(
        pl.kernel,
        out_type=jax.ShapeDtypeStruct((E, F), jnp.float32),
        mesh=_sc_mesh(),
        scratch_types=[
            pltpu.VMEM((K, B), jnp.int32),
            pltpu.VMEM((K, B), jnp.int32),
            [pltpu.VMEM((B, F), jnp.float32)] * 2,
            [pltpu.VMEM((B, F), jnp.float32)] * 2,
            [pltpu.VMEM((B, F), jnp.float32)] * 2,
            [pltpu.SemaphoreType.DMA] * 2,
            [pltpu.SemaphoreType.DMA] * 2,
        ],
    )
    def k(x3_hbm, x4_hbm, idxs_hbm, idxd_hbm, s_hbm, idxs_v, idxd_v,
          b3, b4, bo, sg, sw):
        wid = lax.axis_index("s") * NC + lax.axis_index("c")
        base = wid * EPW
        pltpu.sync_copy(idxs_hbm.at[wid], idxs_v)
        pltpu.sync_copy(idxd_hbm.at[wid], idxd_v)

        def gather(j, t):
            pltpu.async_copy(x3_hbm.at[idxs_v.at[j]], b3[t], sg[t])
            pltpu.async_copy(x4_hbm.at[idxd_v.at[j]], b4[t], sg[t])

        def process(j, t):
            pltpu.make_async_copy(x3_hbm.at[idxs_v.at[0]], b3[t], sg[t]).wait()
            pltpu.make_async_copy(x4_hbm.at[idxd_v.at[0]], b4[t], sg[t]).wait()

            @pl.when(j >= 2)
            def _():
                pltpu.make_async_copy(bo[t], s_hbm.at[pl.ds(0, B)],
                                      sw[t]).wait()

            def row(i, _):
                for l in range(F // 16):
                    sl = pl.ds(l * 16, 16)
                    bo[t][i, sl] = b3[t][i, sl] + b4[t][i, sl]
                return 0

            lax.fori_loop(0, B, row, 0)
            pltpu.async_copy(bo[t], s_hbm.at[pl.ds(base + j * B, B)], sw[t])

            @pl.when(j + 2 < K)
            def _():
                gather(j + 2, t)

        gather(0, 0)
        gather(1, 1)

        def pair(p, _):
            process(2 * p, 0)
            process(2 * p + 1, 1)
            return 0

        lax.fori_loop(0, K // 2, pair, 0)
        process(jnp.int32(K - 1), 0)
        pltpu.make_async_copy(bo[0], s_hbm.at[pl.ds(0, B)], sw[0]).wait()
        pltpu.make_async_copy(bo[1], s_hbm.at[pl.ds(0, B)], sw[1]).wait()

    return k(x3, x4, idx_s3, idx_d3)


def _sc_gate_scatter(x2z, gate, idx_s2, idx_d2, zeros_n, order_dep):
    @functools.partial(
        pl.kernel,
        out_type=jax.ShapeDtypeStruct((NC, NPB2, F), jnp.float32),
        mesh=_sc_mesh(),
        scratch_types=[
            pltpu.VMEM((K2, B2), jnp.int32),
            pltpu.VMEM((K2, B2), jnp.int32),
            [pltpu.VMEM((B2, F), jnp.float32)] * 2,
            [pltpu.VMEM((B2, F), jnp.float32)] * 2,
            [pltpu.VMEM((B2, F), jnp.float32)] * 2,
            pltpu.VMEM_SHARED((NPB2, F), jnp.float32),
            [pltpu.SemaphoreType.DMA] * 2,
            [pltpu.SemaphoreType.DMA] * 2,
        ],
    )
    def k(x2_hbm, g_hbm, idxs_hbm, idxd_hbm, z_hbm, dep_hbm, acc_hbm,
          idxs_v, idxd_v, bx, bw, bs, acc_sh, sg, ss):
        del dep_hbm
        cid = lax.axis_index("c")
        sid = lax.axis_index("s")
        coff = cid * NPB
        rows = pl.ds(sid * NPT2, NPT2)
        pltpu.sync_copy(z_hbm.at[rows], acc_sh.at[rows])
        plsc.subcore_barrier()

        def phase(h):
            base = sid * EPT + h * (K2 * B2)
            pltpu.sync_copy(idxs_hbm.at[sid, h], idxs_v)
            pltpu.sync_copy(idxd_hbm.at[sid, h], idxd_v)

            def adj(r, _):
                for l in range(B2 // 16):
                    sl = pl.ds(l * 16, 16)
                    sv = idxs_v[r, sl]
                    r0 = sv - coff
                    valid = (r0 >= 0) & (r0 < NPB)
                    idxs_v[r, sl] = jnp.where(valid, r0,
                                              NPB + (sv & (NPB2 - NPB - 1)))
                return 0

            lax.fori_loop(0, K2, adj, 0)

            def gather(j, t):
                pltpu.async_copy(x2_hbm.at[idxd_v.at[j]], bx[t], sg[t])
                pltpu.async_copy(g_hbm.at[pl.ds(base + j * B2, B2)],
                                 bw[t], sg[t])

            def process(j, t):
                pltpu.make_async_copy(x2_hbm.at[idxd_v.at[0]],
                                      bx[t], sg[t]).wait()
                pltpu.make_async_copy(g_hbm.at[pl.ds(0, B2)],
                                      bw[t], sg[t]).wait()

                @pl.when(j >= 2)
                def _():
                    pltpu.make_async_copy(bs[t], acc_sh.at[idxs_v.at[0]],
                                          ss[t]).wait()

                def row(i, _):
                    for l in range(F // 16):
                        sl = pl.ds(l * 16, 16)
                        bs[t][i, sl] = bx[t][i, sl] * bw[t][i, sl]
                    return 0

                lax.fori_loop(0, B2, row, 0)

                @pl.when(j + 2 < K2)
                def _():
                    gather(j + 2, t)

                pltpu.async_copy(bs[t], acc_sh.at[idxs_v.at[j]],
                                 ss[t], add=True)

            gather(0, 0)
            gather(1, 1)

            def pair(p, _):
                process(2 * p, 0)
                process(2 * p + 1, 1)
                return 0

            lax.fori_loop(0, K2 // 2, pair, 0)
            process(jnp.int32(K2 - 1), 0)
            pltpu.make_async_copy(bs[0], acc_sh.at[idxs_v.at[0]],
                                  ss[0]).wait()
            pltpu.make_async_copy(bs[1], acc_sh.at[idxs_v.at[0]],
                                  ss[1]).wait()

        for h in range(NH):
            phase(h)
        plsc.subcore_barrier()
        pltpu.sync_copy(acc_sh.at[rows], acc_hbm.at[cid, rows])

    return k(x2z, gate, idx_s2, idx_d2, zeros_n, order_dep)


def _tc_init_node(x, W, b):
    def body(x_ref, w_ref, b_ref, o_ref):
        h = jnp.dot(x_ref[...], w_ref[...],
                    preferred_element_type=jnp.float32) + b_ref[...]
        o_ref[...] = h * jax.nn.sigmoid(h)

    return pl.pallas_call(
        body,
        out_shape=jax.ShapeDtypeStruct((N, F), jnp.float32),
    )(x, W, b.reshape(1, F))


def _tc_init_edge(ea, W, b):
    def body(ea_ref, w_ref, b_ref, o_ref, g_ref):
        h = jnp.dot(ea_ref[...], w_ref[...],
                    preferred_element_type=jnp.float32) + b_ref[...]
        o = h * jax.nn.sigmoid(h)
        o_ref[...] = o
        g_ref[...] = jax.nn.sigmoid(o)

    ef = ea.shape[1]
    return pl.pallas_call(
        body,
        grid=(GE,),
        in_specs=[
            pl.BlockSpec((BE, ef), lambda i: (i, 0)),
            pl.BlockSpec((ef, F), lambda i: (0, 0)),
            pl.BlockSpec((1, F), lambda i: (0, 0)),
        ],
        out_specs=(
            pl.BlockSpec((BE, F), lambda i: (i, 0)),
            pl.BlockSpec((BE, F), lambda i: (i, 0)),
        ),
        out_shape=(
            jax.ShapeDtypeStruct((E, F), jnp.float32),
            jax.ShapeDtypeStruct((E, F), jnp.float32),
        ),
    )(ea, W, b.reshape(1, F))


def _tc_node_mm(x, Wcat, bcat):
    def body(x_ref, w_ref, b_ref, o1, o2, o4, o3):
        h = jnp.dot(x_ref[...], w_ref[...],
                    preferred_element_type=jnp.float32) + b_ref[...]
        o1[...] = h[:, 0 * F:1 * F]
        o2[...] = h[:, 1 * F:2 * F]
        o4[...] = h[:, 2 * F:3 * F]
        o3[...] = h[:, 3 * F:4 * F]

    sh = jax.ShapeDtypeStruct((N, F), jnp.float32)
    return pl.pallas_call(
        body,
        out_shape=(sh, sh, sh, sh),
    )(x, Wcat, bcat.reshape(1, 4 * F))


def _tc_edge_pass1(w, s, E0Wi, E0bi):
    def body(w_ref, s_ref, ew_ref, eb_ref, st_ref):
        h = jnp.dot(w_ref[...], ew_ref[...],
                    preferred_element_type=jnp.float32)
        h = h + eb_ref[...] + s_ref[...]

        @pl.when(pl.program_id(0) == 0)
        def _():
            st_ref[...] = jnp.zeros_like(st_ref)

        st_ref[0:1, :] += jnp.sum(h, axis=0, keepdims=True)
        st_ref[1:2, :] += jnp.sum(h * h, axis=0, keepdims=True)

    return pl.pallas_call(
        body,
        grid=(GE,),
        in_specs=[
            pl.BlockSpec((BE, F), lambda i: (i, 0)),
            pl.BlockSpec((BE, F), lambda i: (i, 0)),
            pl.BlockSpec((F, F), lambda i: (0, 0)),
            pl.BlockSpec((1, F), lambda i: (0, 0)),
        ],
        out_specs=pl.BlockSpec((8, F), lambda i: (0, 0)),
        out_shape=jax.ShapeDtypeStruct((8, F), jnp.float32),
    )(w, s, E0Wi, E0bi.reshape(1, F))


def _tc_inv_deg(cnt):
    def body(c_ref, o_ref):
        c = jnp.concatenate([c_ref[0, :NPB], c_ref[1, :N - NPB]], axis=0)
        o_ref[...] = 1.0 / jnp.maximum(c, 1.0)

    return pl.pallas_call(
        body,
        out_shape=jax.ShapeDtypeStruct((N, F), jnp.float32),
    )(cnt)


def _tc_node_update(x0, x1, acc, inv, g, b):
    def body(x0_ref, x1_ref, a_ref, i_ref, g_ref, b_ref, o_ref):
        asum = jnp.concatenate(
            [a_ref[0, :NPB], a_ref[1, :N - NPB]], axis=0)
        t = x1_ref[...] + asum * i_ref[...]
        m = jnp.mean(t, axis=0, keepdims=True)
        v = jnp.mean(t * t, axis=0, keepdims=True) - m * m
        y = (t - m) * lax.rsqrt(v + 1e-5) * g_ref[...] + b_ref[...]
        o_ref[...] = x0_ref[...] + y * jax.nn.sigmoid(y)

    return pl.pallas_call(
        body,
        out_shape=jax.ShapeDtypeStruct((N, F), jnp.float32),
    )(x0, x1, acc, inv, g.reshape(1, F), b.reshape(1, F))


def _tc_edge_update(w0, s, E0Wi, E0bi, st, g, b, want_gate):
    def body(w_ref, s_ref, ew_ref, eb_ref, st_ref, g_ref, b_ref,
             o_ref, *rest):
        h = jnp.dot(w_ref[...], ew_ref[...],
                    preferred_element_type=jnp.float32)
        h = h + eb_ref[...] + s_ref[...]
        m = st_ref[0:1, :] * (1.0 / E)
        v = st_ref[1:2, :] * (1.0 / E) - m * m
        y = (h - m) * lax.rsqrt(v + 1e-5) * g_ref[...] + b_ref[...]
        o = w_ref[...] + y * jax.nn.sigmoid(y)
        o_ref[...] = o
        if rest:
            rest[0][...] = jax.nn.sigmoid(o)

    out_specs = [pl.BlockSpec((BE, F), lambda i: (i, 0))]
    out_shape = [jax.ShapeDtypeStruct((E, F), jnp.float32)]
    if want_gate:
        out_specs.append(pl.BlockSpec((BE, F), lambda i: (i, 0)))
        out_shape.append(jax.ShapeDtypeStruct((E, F), jnp.float32))
    res = pl.pallas_call(
        body,
        grid=(GE,),
        in_specs=[
            pl.BlockSpec((BE, F), lambda i: (i, 0)),
            pl.BlockSpec((BE, F), lambda i: (i, 0)),
            pl.BlockSpec((F, F), lambda i: (0, 0)),
            pl.BlockSpec((1, F), lambda i: (0, 0)),
            pl.BlockSpec((8, F), lambda i: (0, 0)),
            pl.BlockSpec((1, F), lambda i: (0, 0)),
            pl.BlockSpec((1, F), lambda i: (0, 0)),
        ],
        out_specs=tuple(out_specs),
        out_shape=tuple(out_shape),
    )(w0, s, E0Wi, E0bi.reshape(1, F), st, g.reshape(1, F),
      b.reshape(1, F))
    return res if want_gate else (res[0], None)


def kernel(x, edge_index, edge_attr, v_lin0_W, v_lin0_b, V1_W, V1_b, V2_W,
           V2_b, V3_W, V3_b, V4_W, V4_b, vbn_g, vbn_b, e_lin0_W, e_lin0_b,
           E0_W, E0_b, ebn_g, ebn_b):
    src = edge_index[0]
    dst = edge_index[1]
    idx_s3 = src.reshape(NW, K, B)
    idx_d3 = dst.reshape(NW, K, B)
    idx_s2 = src.reshape(NS, NH, K2, B2)
    idx_d2 = dst.reshape(NS, NH, K2, B2)
    zeros_b2 = jnp.zeros((NPB2, F), jnp.float32)
    ones_z = jnp.ones((N, F), jnp.float32)
    ones_e = jnp.ones((E, F), jnp.float32)

    Wcat = jnp.concatenate([V1_W, V2_W, V4_W, V3_W], axis=2)
    bcat = jnp.concatenate([V1_b, V2_b, V4_b, V3_b], axis=1)

    dep0 = jnp.zeros((8, F), jnp.float32)
    cnt = _sc_gate_scatter(ones_z, ones_e, idx_s2, idx_d2, zeros_b2, dep0)
    inv = _tc_inv_deg(cnt)

    xc = _tc_init_node(x, v_lin0_W, v_lin0_b)
    wc, gc = _tc_init_edge(edge_attr, e_lin0_W, e_lin0_b)

    x1, x2z, x4, x3 = _tc_node_mm(xc, Wcat[0], bcat[0])
    s = _sc_gather_sum(x3, x4, idx_s3, idx_d3)
    acc = _sc_gate_scatter(x2z, gc, idx_s2, idx_d2, zeros_b2, s[:8])
    for i in range(DEPTH):
        st = _tc_edge_pass1(wc, s, E0_W[i], E0_b[i])
        xc = _tc_node_update(xc, x1, acc, inv, vbn_g[i], vbn_b[i])
        if i + 1 < DEPTH:
            x1, x2z, x4, x3 = _tc_node_mm(xc, Wcat[i + 1], bcat[i + 1])
            sn = _sc_gather_sum(x3, x4, idx_s3, idx_d3)
            wc, gc = _tc_edge_update(wc, s, E0_W[i], E0_b[i], st,
                                     ebn_g[i], ebn_b[i], want_gate=True)
            acc = _sc_gate_scatter(x2z, gc, idx_s2, idx_d2, zeros_b2, sn[:8])
            s = sn
        else:
            wc, _ = _tc_edge_update(wc, s, E0_W[i], E0_b[i], st,
                                    ebn_g[i], ebn_b[i], want_gate=False)
    return (xc, wc)
